# baseline (speedup 1.0000x reference)
import functools

import jax
import jax.numpy as jnp
from jax import lax
from jax.experimental import pallas as pl
from jax.experimental.pallas import tpu as pltpu
from jax.experimental.pallas import tpu_sc as plsc

VOCAB = 256
EMBED = 512

NUM_CORES = 2
NUM_SUBCORES = 16
NW = NUM_CORES * NUM_SUBCORES

B_TOTAL = 4 * 8192
NCOLG = 2
COLS = EMBED // NCOLG
NROWG = NW // NCOLG
ROWS = B_TOTAL // NROWG
CHUNK = 32
NCHUNK = ROWS // CHUNK
GRP = 4
NGRP = NCHUNK // GRP
LANES = 16


def _make_gather():
    mesh = plsc.VectorSubcoreMesh(core_axis_name="c", subcore_axis_name="s")

    @functools.partial(
        pl.kernel,
        mesh=mesh,
        compiler_params=pltpu.CompilerParams(
            needs_layout_passes=False, disable_bounds_checks=True),
        out_type=jax.ShapeDtypeStruct((B_TOTAL, EMBED), jnp.float32),
        scratch_types=[
            pltpu.VMEM((ROWS // 2 * LANES,), jnp.int32),
            pltpu.VMEM((ROWS,), jnp.int32),
            pltpu.VMEM((VOCAB * COLS,), jnp.float32),
            [pltpu.VMEM((CHUNK, COLS), jnp.float32) for _ in range(3)],
            pltpu.VMEM((CHUNK, COLS), jnp.float32),
            pltpu.SemaphoreType.DMA,
            pltpu.SemaphoreType.DMA,
            pltpu.SemaphoreType.DMA,
        ],
    )
    def gather_kernel(idxb_hbm, idxu_hbm, tflat_hbm, t2d_hbm, out_hbm,
                      idx_b, idx_u, table_v, fbufs, sbuf, sem_t, sem_g,
                      sem_w):
        wid = lax.axis_index("s") * NUM_CORES + lax.axis_index("c")
        rowg = wid // NCOLG
        colg = wid % NCOLG
        row_base = rowg * ROWS

        pltpu.async_copy(
            tflat_hbm.at[pl.ds(colg * (VOCAB * COLS), VOCAB * COLS)],
            table_v, sem_t)
        pltpu.sync_copy(idxu_hbm.at[colg, pl.ds(row_base, ROWS)], idx_u)
        pltpu.sync_copy(
            idxb_hbm.at[pl.ds(row_base * LANES, ROWS // 2 * LANES)], idx_b)
        pltpu.make_async_copy(
            tflat_hbm.at[pl.ds(colg * (VOCAB * COLS), VOCAB * COLS)],
            table_v, sem_t).wait()

        iota16 = lax.iota(jnp.int32, LANES)

        def fill(j, buf):
            jh = j % (NCHUNK // 2)

            @plsc.parallel_loop(0, CHUNK, step=1, unroll=8)
            def rowfn(r):
                row_off = idx_b[pl.ds((jh * CHUNK + r) * LANES, LANES)]

                @plsc.parallel_loop(0, COLS // LANES, step=1, unroll=16,
                                    carry=iota16)
                def colfn(k, colv):
                    vals = plsc.load_gather(table_v, [row_off + colv])
                    buf[r, pl.ds(k * LANES, LANES)] = vals
                    return colv + LANES

        def gather(j):
            return pltpu.async_copy(
                t2d_hbm.at[idx_u.at[pl.ds(j * CHUNK, CHUNK)]], sbuf, sem_g)

        def write(j, buf):
            return pltpu.async_copy(
                buf,
                out_hbm.at[pl.ds(row_base + j * CHUNK, CHUNK),
                           pl.ds(colg * COLS, COLS)],
                sem_w)

        def wait_write():
            pltpu.make_async_copy(
                fbufs[0],
                out_hbm.at[pl.ds(row_base, CHUNK),
                           pl.ds(colg * COLS, COLS)],
                sem_w).wait()

        def wait_gather():
            pltpu.make_async_copy(
                t2d_hbm.at[idx_u.at[pl.ds(0, CHUNK)]], sbuf, sem_g).wait()

        def group(g, waits):
            gather(GRP * g)
            if waits:
                wait_write()
                wait_write()
            fill(GRP * g + 1, fbufs[0])
            write(GRP * g + 1, fbufs[0])
            wait_gather()
            write(GRP * g, sbuf)
            if waits:
                wait_write()
            fill(GRP * g + 2, fbufs[1])
            write(GRP * g + 2, fbufs[1])
            if waits:
                wait_write()
            fill(GRP * g + 3, fbufs[2])
            write(GRP * g + 3, fbufs[2])

        group(0, False)

        def body(g, _):
            @pl.when(g == NGRP // 2)
            def _():
                pltpu.sync_copy(
                    idxb_hbm.at[pl.ds(
                        (row_base + ROWS // 2) * LANES, ROWS // 2 * LANES)],
                    idx_b)

            group(g, True)
            return 0

        lax.fori_loop(1, NGRP, body, 0)
        for _ in range(GRP):
            wait_write()

    return gather_kernel


_gather = _make_gather()


@jax.jit
def kernel(x, embedding):
    idx = x.reshape(B_TOTAL).astype(jnp.int32)
    idx_b = jnp.broadcast_to(
        (idx * COLS)[:, None], (B_TOTAL, LANES)).reshape(-1)
    idx_u = jnp.stack([idx, idx + VOCAB])
    table2d = (
        embedding.reshape(VOCAB, NCOLG, COLS)
        .transpose(1, 0, 2)
        .reshape(NCOLG * VOCAB, COLS)
    )
    tableflat = table2d.reshape(-1)
    out = _gather(idx_b, idx_u, tableflat, table2d)
    return out.reshape(x.shape + (EMBED,))


# speedup vs baseline: 1.0989x; 1.0989x over previous
import functools

import jax
import jax.numpy as jnp
from jax import lax
from jax.experimental import pallas as pl
from jax.experimental.pallas import tpu as pltpu
from jax.experimental.pallas import tpu_sc as plsc

VOCAB = 256
EMBED = 512

NUM_CORES = 2
NUM_SUBCORES = 16
NW = NUM_CORES * NUM_SUBCORES

B_TOTAL = 4 * 8192
NCOLG = 2
COLS = EMBED // NCOLG
NROWG = NW // NCOLG
ROWS = B_TOTAL // NROWG
CHUNK = 32
NCHUNK = ROWS // CHUNK
GRP = 8
NGRP = NCHUNK // GRP
LANES = 16


def _make_gather():
    mesh = plsc.VectorSubcoreMesh(core_axis_name="c", subcore_axis_name="s")

    @functools.partial(
        pl.kernel,
        mesh=mesh,
        compiler_params=pltpu.CompilerParams(
            needs_layout_passes=False, disable_bounds_checks=True),
        out_type=jax.ShapeDtypeStruct((B_TOTAL, EMBED), jnp.float32),
        scratch_types=[
            pltpu.VMEM((ROWS // 2 * LANES,), jnp.int32),
            pltpu.VMEM((ROWS,), jnp.int32),
            pltpu.VMEM((VOCAB * COLS,), jnp.float32),
            [pltpu.VMEM((CHUNK, COLS), jnp.float32) for _ in range(4)],
            pltpu.VMEM((CHUNK, COLS), jnp.float32),
            pltpu.SemaphoreType.DMA,
            pltpu.SemaphoreType.DMA,
            pltpu.SemaphoreType.DMA,
        ],
    )
    def gather_kernel(idxb_hbm, idxu_hbm, tflat_hbm, t2d_hbm, out_hbm,
                      idx_b, idx_u, table_v, fbufs, sbuf, sem_t, sem_g,
                      sem_w):
        wid = lax.axis_index("s") * NUM_CORES + lax.axis_index("c")
        rowg = wid // NCOLG
        colg = wid % NCOLG
        row_base = rowg * ROWS

        pltpu.async_copy(
            tflat_hbm.at[pl.ds(colg * (VOCAB * COLS), VOCAB * COLS)],
            table_v, sem_t)
        pltpu.sync_copy(idxu_hbm.at[colg, pl.ds(row_base, ROWS)], idx_u)
        pltpu.sync_copy(
            idxb_hbm.at[pl.ds(row_base * LANES, ROWS // 2 * LANES)], idx_b)
        pltpu.make_async_copy(
            tflat_hbm.at[pl.ds(colg * (VOCAB * COLS), VOCAB * COLS)],
            table_v, sem_t).wait()

        iota16 = lax.iota(jnp.int32, LANES)

        def fill(j, buf):
            jh = j % (NCHUNK // 2)

            @plsc.parallel_loop(0, CHUNK, step=1, unroll=8)
            def rowfn(r):
                row_off = idx_b[pl.ds((jh * CHUNK + r) * LANES, LANES)]

                @plsc.parallel_loop(0, COLS // LANES, step=1, unroll=16,
                                    carry=iota16)
                def colfn(k, colv):
                    vals = plsc.load_gather(table_v, [row_off + colv])
                    buf[r, pl.ds(k * LANES, LANES)] = vals
                    return colv + LANES

        def gather(j):
            return pltpu.async_copy(
                t2d_hbm.at[idx_u.at[pl.ds(j * CHUNK, CHUNK)]], sbuf, sem_g)

        def write(j, buf):
            return pltpu.async_copy(
                buf,
                out_hbm.at[pl.ds(row_base + j * CHUNK, CHUNK),
                           pl.ds(colg * COLS, COLS)],
                sem_w)

        def wait_write():
            pltpu.make_async_copy(
                fbufs[0],
                out_hbm.at[pl.ds(row_base, CHUNK),
                           pl.ds(colg * COLS, COLS)],
                sem_w).wait()

        def wait_gather():
            pltpu.make_async_copy(
                t2d_hbm.at[idx_u.at[pl.ds(0, CHUNK)]], sbuf, sem_g).wait()

        STEADY_WAITS = (2, 1, 1, 0, 1, 1, 1)
        FIRST_WAITS = (0, 0, 0, 0, 1, 1, 1)
        FILL_BUFS = (0, 1, 2, 3, 0, 1, 2)

        def group(g, steady):
            pre = STEADY_WAITS if steady else FIRST_WAITS
            for i in range(GRP - 1):
                if i == 4:
                    if steady:
                        wait_write()
                    gather(GRP * g)
                for _ in range(pre[i]):
                    wait_write()
                j = GRP * g + 1 + i
                fill(j, fbufs[FILL_BUFS[i]])
                write(j, fbufs[FILL_BUFS[i]])
            wait_gather()
            write(GRP * g, sbuf)

        group(0, False)

        def body(g, _):
            @pl.when(g == NGRP // 2)
            def _():
                pltpu.sync_copy(
                    idxb_hbm.at[pl.ds(
                        (row_base + ROWS // 2) * LANES, ROWS // 2 * LANES)],
                    idx_b)

            group(g, True)
            return 0

        lax.fori_loop(1, NGRP, body, 0)
        for _ in range(5):
            wait_write()

    return gather_kernel


_gather = _make_gather()


@jax.jit
def kernel(x, embedding):
    idx = x.reshape(B_TOTAL).astype(jnp.int32)
    idx_b = jnp.broadcast_to(
        (idx * COLS)[:, None], (B_TOTAL, LANES)).reshape(-1)
    idx_u = jnp.stack([idx, idx + VOCAB])
    table2d = (
        embedding.reshape(VOCAB, NCOLG, COLS)
        .transpose(1, 0, 2)
        .reshape(NCOLG * VOCAB, COLS)
    )
    tableflat = table2d.reshape(-1)
    out = _gather(idx_b, idx_u, tableflat, table2d)
    return out.reshape(x.shape + (EMBED,))


# speedup vs baseline: 1.2381x; 1.1266x over previous
import functools

import jax
import jax.numpy as jnp
from jax import lax
from jax.experimental import pallas as pl
from jax.experimental.pallas import tpu as pltpu
from jax.experimental.pallas import tpu_sc as plsc

VOCAB = 256
EMBED = 512

NUM_CORES = 2
NUM_SUBCORES = 16
NW = NUM_CORES * NUM_SUBCORES

B_TOTAL = 4 * 8192
NCOLG = 2
COLS = EMBED // NCOLG
NROWG = NW // NCOLG
ROWS = B_TOTAL // NROWG
CHUNK = 64
NCHUNK = ROWS // CHUNK
NBUF = 2
LANES = 16


def _make_gather():
    mesh = plsc.VectorSubcoreMesh(core_axis_name="c", subcore_axis_name="s")

    @functools.partial(
        pl.kernel,
        mesh=mesh,
        compiler_params=pltpu.CompilerParams(
            needs_layout_passes=False, disable_bounds_checks=True),
        out_type=jax.ShapeDtypeStruct((B_TOTAL, EMBED), jnp.float32),
        scratch_types=[
            pltpu.VMEM((ROWS // 2 * LANES,), jnp.int32),
            pltpu.VMEM((VOCAB * COLS,), jnp.float32),
            [pltpu.VMEM((CHUNK, COLS), jnp.float32) for _ in range(NBUF)],
            pltpu.SemaphoreType.DMA,
            pltpu.SemaphoreType.DMA,
        ],
    )
    def gather_kernel(idxb_hbm, table_hbm, out_hbm, idx_b, table_v, bufs,
                      sem_t, sem_w):
        wid = lax.axis_index("s") * NUM_CORES + lax.axis_index("c")
        rowg = wid // NCOLG
        colg = wid % NCOLG
        row_base = rowg * ROWS

        pltpu.async_copy(
            table_hbm.at[pl.ds(colg * (VOCAB * COLS), VOCAB * COLS)],
            table_v, sem_t)
        pltpu.sync_copy(
            idxb_hbm.at[pl.ds(row_base * LANES, ROWS // 2 * LANES)], idx_b)
        pltpu.make_async_copy(
            table_hbm.at[pl.ds(colg * (VOCAB * COLS), VOCAB * COLS)],
            table_v, sem_t).wait()

        iota16 = lax.iota(jnp.int32, LANES)

        def fill(j, buf):
            @plsc.parallel_loop(0, CHUNK, step=1, unroll=8)
            def rowfn(r):
                row_off = idx_b[
                    pl.ds(((j % (NCHUNK // 2)) * CHUNK + r) * LANES, LANES)]

                @plsc.parallel_loop(0, COLS // LANES, step=1, unroll=16,
                                    carry=iota16)
                def colfn(k, colv):
                    vals = plsc.load_gather(table_v, [row_off + colv])
                    buf[r, pl.ds(k * LANES, LANES)] = vals
                    return colv + LANES

        def write(j, buf):
            return pltpu.async_copy(
                buf,
                out_hbm.at[pl.ds(row_base + j * CHUNK, CHUNK),
                           pl.ds(colg * COLS, COLS)],
                sem_w)

        def wait_write(buf):
            pltpu.make_async_copy(
                buf,
                out_hbm.at[pl.ds(row_base, CHUNK),
                           pl.ds(colg * COLS, COLS)],
                sem_w).wait()

        for j in range(NBUF):
            fill(j, bufs[j])
            write(j, bufs[j])

        half_g = NCHUNK // 2 // NBUF

        def pair(g, _):
            @pl.when(g == half_g)
            def _():
                pltpu.sync_copy(
                    idxb_hbm.at[pl.ds(
                        (row_base + ROWS // 2) * LANES, ROWS // 2 * LANES)],
                    idx_b)

            for b in range(NBUF):
                j = g * NBUF + b
                wait_write(bufs[b])
                fill(j, bufs[b])
                write(j, bufs[b])
            return 0

        lax.fori_loop(1, NCHUNK // NBUF, pair, 0)
        for b in range(NBUF):
            wait_write(bufs[b])

    return gather_kernel


_gather = _make_gather()


@jax.jit
def kernel(x, embedding):
    idx = x.reshape(B_TOTAL).astype(jnp.int32)
    idx_b = jnp.broadcast_to(
        (idx * COLS)[:, None], (B_TOTAL, LANES)).reshape(-1)
    table = (
        embedding.reshape(VOCAB, NCOLG, COLS)
        .transpose(1, 0, 2)
        .reshape(NCOLG * VOCAB * COLS)
    )
    out = _gather(idx_b, table)
    return out.reshape(x.shape + (EMBED,))
